# baseline (speedup 1.0000x reference)
import jax
import jax.numpy as jnp
from jax import lax
from jax.experimental import pallas as pl
from jax.experimental.pallas import tpu as pltpu, tpu_sc as plsc

_NUM = 4096
_CHILD = 65536
_CPN = 16
_BATCH = 512
_NC, _NS = 2, 16
_NW = _NC * _NS
_CH_W = _CHILD // _NW
_NODES_W = _NUM // _NW
_NBLK_N = _NODES_W // _CPN
_RB = 8
_NBLK = _BATCH // _RB


def _vlog(x):
    ix = lax.bitcast_convert_type(x, jnp.int32)
    e = lax.shift_right_arithmetic(ix - 0x3F3504F3, 23)
    m = lax.bitcast_convert_type(ix - lax.shift_left(e, 23), jnp.float32)
    s = (m - 1.0) / (m + 1.0)
    z = s * s
    p = z * (jnp.float32(1.0 / 3.0) + z * (jnp.float32(0.2)
             + z * jnp.float32(1.0 / 7.0)))
    return e.astype(jnp.float32) * jnp.float32(0.6931471805599453) \
        + 2.0 * (s + s * p)


def _sc_body(x_hbm, lp_hbm, out_hbm, lp_v, lpt_v, lse_v, xb0, xb1, acc_v,
             sem0, sem1):
    wid = lax.axis_index("s") * _NC + lax.axis_index("c")
    ch0 = wid * _CH_W
    col0 = wid * _NODES_W
    bi16 = lax.iota(jnp.int32, 16) * 16

    pltpu.sync_copy(lp_hbm.at[pl.ds(ch0, _CH_W)], lp_v)
    for k in range(_NBLK_N):
        acc = None
        for c in range(_CPN):
            g = plsc.load_gather(lp_v, [bi16 + (k * 256 + c)])
            lpt_v[c, pl.ds(k * _CPN, _CPN)] = g
            e = jnp.exp(g)
            acc = e if acc is None else acc + e
        lse_v[pl.ds(k * _CPN, _CPN)] = _vlog(acc)

    def _start(buf, sem, blk):
        pltpu.make_async_copy(
            x_hbm.at[pl.ds(blk * _RB, _RB), pl.ds(ch0, _CH_W)], buf, sem
        ).start()

    def _wait(buf, sem):
        pltpu.make_async_copy(
            x_hbm.at[pl.ds(0, _RB), pl.ds(ch0, _CH_W)], buf, sem
        ).wait()

    def _compute(buf, blk):
        row0 = blk * _RB

        def node_body(n0, _):
            nbase = n0 * (_CPN * _CPN)
            ncol = n0 * _CPN
            lpv = [lpt_v[c, pl.ds(ncol, _CPN)] for c in range(_CPN)]
            for b in range(_RB):
                row = jnp.full((16,), b, jnp.int32)
                acc = None
                for c in range(_CPN):
                    xg = plsc.load_gather(buf, [row, bi16 + nbase + c])
                    t = jnp.exp(xg + lpv[c])
                    acc = t if acc is None else acc + t
                acc_v[row0 + b, pl.ds(ncol, _CPN)] = acc
            return 0
        lax.fori_loop(0, _NBLK_N, node_body, 0)

    _start(xb0, sem0, 0)

    def blk_body(i2, _):
        blk0 = i2 * 2
        _start(xb1, sem1, blk0 + 1)
        _wait(xb0, sem0)
        _compute(xb0, blk0)

        @pl.when(i2 < _NBLK // 2 - 1)
        def _():
            _start(xb0, sem0, blk0 + 2)

        _wait(xb1, sem1)
        _compute(xb1, blk0 + 1)
        return 0
    lax.fori_loop(0, _NBLK // 2, blk_body, 0)

    def fin_body(b, _):
        for k in range(_NBLK_N):
            sl = pl.ds(k * _CPN, _CPN)
            acc_v[b, sl] = _vlog(acc_v[b, sl]) - lse_v[sl]
        return 0
    lax.fori_loop(0, _BATCH, fin_body, 0)
    pltpu.sync_copy(acc_v, out_hbm.at[:, pl.ds(col0, _NODES_W)])


def kernel(input, logparams):
    mesh = plsc.VectorSubcoreMesh(core_axis_name="c", subcore_axis_name="s")
    f = pl.kernel(
        _sc_body,
        out_type=jax.ShapeDtypeStruct((_BATCH, _NUM), jnp.float32),
        mesh=mesh,
        compiler_params=pltpu.CompilerParams(needs_layout_passes=False),
        scratch_types=[
            pltpu.VMEM((_CH_W,), jnp.float32),
            pltpu.VMEM((_CPN, _NODES_W), jnp.float32),
            pltpu.VMEM((_NODES_W,), jnp.float32),
            pltpu.VMEM((_RB, _CH_W), jnp.float32),
            pltpu.VMEM((_RB, _CH_W), jnp.float32),
            pltpu.VMEM((_BATCH, _NODES_W), jnp.float32),
            pltpu.SemaphoreType.DMA,
            pltpu.SemaphoreType.DMA,
        ],
    )
    return f(input, logparams)


# speedup vs baseline: 1.1270x; 1.1270x over previous
import jax
import jax.numpy as jnp
from jax import lax
from jax.experimental import pallas as pl
from jax.experimental.pallas import tpu as pltpu, tpu_sc as plsc

_NUM = 4096
_CHILD = 65536
_CPN = 16
_BATCH = 512
_NC, _NS = 2, 16
_NW = _NC * _NS
_CH_W = _CHILD // _NW
_NODES_W = _NUM // _NW
_NBLK_N = _NODES_W // _CPN
_RB = 8
_NBLK = _BATCH // _RB


def _vlog(x):
    ix = lax.bitcast_convert_type(x, jnp.int32)
    e = lax.shift_right_arithmetic(ix - 0x3F3504F3, 23)
    m = lax.bitcast_convert_type(ix - lax.shift_left(e, 23), jnp.float32)
    s = (m - 1.0) / (m + 1.0)
    z = s * s
    p = z * (jnp.float32(1.0 / 3.0) + z * (jnp.float32(0.2)
             + z * jnp.float32(1.0 / 7.0)))
    return e.astype(jnp.float32) * jnp.float32(0.6931471805599453) \
        + 2.0 * (s + s * p)


def _sc_body(x_hbm, lp_hbm, out_hbm, lp_v, lpt_v, lse_v, idx_t, xb0, xb1,
             acc_v, sem0, sem1):
    wid = lax.axis_index("s") * _NC + lax.axis_index("c")
    ch0 = wid * _CH_W
    col0 = wid * _NODES_W
    bi16 = lax.iota(jnp.int32, 16) * 16

    pltpu.sync_copy(lp_hbm.at[pl.ds(ch0, _CH_W)], lp_v)
    for k in range(_NBLK_N):
        acc = None
        for c in range(_CPN):
            idx = bi16 + (k * 256 + c)
            idx_t[k * _CPN + c, :] = idx
            g = plsc.load_gather(lp_v, [idx])
            lpt_v[c, pl.ds(k * _CPN, _CPN)] = g
            e = jnp.exp(g)
            acc = e if acc is None else acc + e
        lse_v[pl.ds(k * _CPN, _CPN)] = _vlog(acc)

    def _start(buf, sem, blk):
        for b in range(_RB):
            pltpu.make_async_copy(
                x_hbm.at[blk * _RB + b, pl.ds(ch0, _CH_W)],
                buf.at[pl.ds(b * _CH_W, _CH_W)], sem
            ).start()

    def _wait(buf, sem):
        pltpu.make_async_copy(
            x_hbm.at[0, pl.ds(0, _RB * _CH_W)], buf, sem
        ).wait()

    def _compute(buf, blk):
        row0 = blk * _RB

        for n0 in range(_NBLK_N):
            ncol = n0 * _CPN
            idxs = [idx_t[ncol + c, :] for c in range(_CPN)]
            lpv = [lpt_v[c, pl.ds(ncol, _CPN)] for c in range(_CPN)]

            @plsc.parallel_loop(0, _RB, 2, unroll=2)
            def pair_body(b):
                off0 = pl.multiple_of(b * _CH_W, _CH_W)
                r0 = buf.at[pl.ds(off0, _CH_W)]
                r1 = buf.at[pl.ds(off0 + _CH_W, _CH_W)]
                a0 = [None] * 4
                a1 = [None] * 4
                for c in range(_CPN):
                    g0 = plsc.load_gather(r0, [idxs[c]])
                    g1 = plsc.load_gather(r1, [idxs[c]])
                    t0 = jnp.exp(g0 + lpv[c])
                    t1 = jnp.exp(g1 + lpv[c])
                    p0, p1 = a0[c % 4], a1[c % 4]
                    a0[c % 4] = t0 if p0 is None else p0 + t0
                    a1[c % 4] = t1 if p1 is None else p1 + t1
                acc_v[row0 + b, pl.ds(ncol, _CPN)] = (
                    (a0[0] + a0[1]) + (a0[2] + a0[3]))
                acc_v[row0 + b + 1, pl.ds(ncol, _CPN)] = (
                    (a1[0] + a1[1]) + (a1[2] + a1[3]))

    _start(xb0, sem0, 0)

    def blk_body(i2, _):
        blk0 = i2 * 2
        _start(xb1, sem1, blk0 + 1)
        _wait(xb0, sem0)
        _compute(xb0, blk0)

        @pl.when(i2 < _NBLK // 2 - 1)
        def _():
            _start(xb0, sem0, blk0 + 2)

        _wait(xb1, sem1)
        _compute(xb1, blk0 + 1)
        return 0
    lax.fori_loop(0, _NBLK // 2, blk_body, 0)

    lses = [lse_v[pl.ds(k * _CPN, _CPN)] for k in range(_NBLK_N)]

    @plsc.parallel_loop(0, _BATCH, 1, unroll=4)
    def fin_body(b):
        for k in range(_NBLK_N):
            sl = pl.ds(k * _CPN, _CPN)
            acc_v[b, sl] = _vlog(acc_v[b, sl]) - lses[k]
    pltpu.sync_copy(acc_v, out_hbm.at[:, pl.ds(col0, _NODES_W)])


def kernel(input, logparams):
    mesh = plsc.VectorSubcoreMesh(core_axis_name="c", subcore_axis_name="s")
    f = pl.kernel(
        _sc_body,
        out_type=jax.ShapeDtypeStruct((_BATCH, _NUM), jnp.float32),
        mesh=mesh,
        compiler_params=pltpu.CompilerParams(needs_layout_passes=False),
        scratch_types=[
            pltpu.VMEM((_CH_W,), jnp.float32),
            pltpu.VMEM((_CPN, _NODES_W), jnp.float32),
            pltpu.VMEM((_NODES_W,), jnp.float32),
            pltpu.VMEM((_NODES_W, _CPN), jnp.int32),
            pltpu.VMEM((_RB * _CH_W,), jnp.float32),
            pltpu.VMEM((_RB * _CH_W,), jnp.float32),
            pltpu.VMEM((_BATCH, _NODES_W), jnp.float32),
            pltpu.SemaphoreType.DMA,
            pltpu.SemaphoreType.DMA,
        ],
    )
    return f(input, logparams)
